# baseline (speedup 1.0000x reference)
import functools
import math

import jax
import jax.numpy as jnp
from jax import lax
from jax.experimental import pallas as pl
from jax.experimental.pallas import tpu as pltpu
from jax.experimental.pallas import tpu_sc as plsc

DMODEL = 64
SCALE = math.sqrt(DMODEL)

_NC = 2
_NS = 16
_NW = _NC * _NS

_LANES = 128
_B0 = 4096
_B1 = 200
_CTILES = _B0 // _LANES
_NBLK = _B1 * _CTILES
_BPW = _NBLK // _NW
_IDX_PER_W = _BPW * _LANES


def _make_lookup():
    mesh = plsc.VectorSubcoreMesh(core_axis_name="c", subcore_axis_name="s")

    @functools.partial(
        pl.kernel,
        out_type=jax.ShapeDtypeStruct((_B1, 8, _CTILES, 8, _LANES),
                                      jnp.float32),
        mesh=mesh,
        scratch_types=(
            [pltpu.VMEM((_IDX_PER_W,), jnp.int32)]
            + [pltpu.VMEM((_LANES, DMODEL), jnp.float32)] * 2
            + [pltpu.VMEM((8, 8, _LANES), jnp.float32)] * 2
            + [pltpu.SemaphoreType.DMA] * 4
        ),
        compiler_params=pltpu.CompilerParams(use_tc_tiling_on_sc=False,
                                             needs_layout_passes=False),
    )
    def lookup(idx_hbm, table_hbm, out_hbm,
               idx_v, rows0, rows1, tb0, tb1, g0, g1, s0, s1):
        rows, tbuf = (rows0, rows1), (tb0, tb1)
        gsem, ssem = (g0, g1), (s0, s1)
        wid = lax.axis_index("s") * _NC + lax.axis_index("c")
        jbase = wid * _BPW
        pltpu.sync_copy(idx_hbm.at[pl.ds(jbase * _LANES, _IDX_PER_W)], idx_v)

        iotas = [lax.iota(jnp.int32, 16) + 16 * k for k in range(8)]

        def start_gather(t, b):
            pltpu.async_copy(
                table_hbm.at[idx_v.at[pl.ds(t * _LANES, _LANES)]],
                rows[b], gsem[b])

        def wait_gather(b):
            pltpu.make_async_copy(
                table_hbm.at[idx_v.at[pl.ds(0, _LANES)]],
                rows[b], gsem[b]).wait()

        def start_store(t, b):
            j = jbase + t
            b1 = j // _CTILES
            c = j % _CTILES
            pltpu.async_copy(tbuf[b], out_hbm.at[b1, :, c], ssem[b])

        def wait_store(b):
            pltpu.make_async_copy(tbuf[b], out_hbm.at[0, :, 0],
                                  ssem[b]).wait()

        def transpose_scale(b):
            rb, tb = rows[b], tbuf[b]

            def body_d(d, carry):
                dsp = jnp.full((16,), d, jnp.int32)
                r = d // 8
                s = d % 8
                for k in range(8):
                    v = plsc.load_gather(rb, [iotas[k], dsp])
                    tb[r, s, pl.ds(16 * k, 16)] = v * SCALE
                return carry

            lax.fori_loop(0, DMODEL, body_d, 0, unroll=False)

        def step(t, b, wait_st, prefetch):
            wait_gather(b)
            if wait_st:
                wait_store(b)
            transpose_scale(b)
            start_store(t, b)
            if prefetch:
                start_gather(t + 2, b)

        start_gather(0, 0)
        start_gather(1, 1)
        step(0, 0, wait_st=False, prefetch=True)
        step(1, 1, wait_st=False, prefetch=True)

        def round_body(i, carry):
            step(2 * i, 0, wait_st=True, prefetch=True)
            step(2 * i + 1, 1, wait_st=True, prefetch=True)
            return carry

        lax.fori_loop(1, _BPW // 2 - 1, round_body, 0, unroll=False)

        step(_BPW - 2, 0, wait_st=True, prefetch=False)
        step(_BPW - 1, 1, wait_st=True, prefetch=False)
        wait_store(0)
        wait_store(1)

    return lookup


def kernel(x, table):
    idx = x.T.reshape(-1).astype(jnp.int32)
    out5 = _make_lookup()(idx, table)
    return out5.transpose(2, 4, 0, 1, 3).reshape(_B0, _B1, DMODEL)


# speedup vs baseline: 1.7566x; 1.7566x over previous
import functools
import math

import jax
import jax.numpy as jnp
from jax import lax
from jax.experimental import pallas as pl
from jax.experimental.pallas import tpu as pltpu
from jax.experimental.pallas import tpu_sc as plsc

DMODEL = 64
SCALE = math.sqrt(DMODEL)

_NC = 2
_NS = 16
_NW = _NC * _NS

_LANES = 128
_B0 = 4096
_B1 = 200
_CTILES = _B0 // _LANES
_NBLK = _B1 * _CTILES
_BPW = _NBLK // _NW
_IDX_PER_W = _BPW * _LANES


def _make_lookup():
    mesh = plsc.VectorSubcoreMesh(core_axis_name="c", subcore_axis_name="s")

    @functools.partial(
        pl.kernel,
        out_type=jax.ShapeDtypeStruct((_B1, 8, _CTILES, 8, _LANES),
                                      jnp.float32),
        mesh=mesh,
        scratch_types=(
            [pltpu.VMEM((_IDX_PER_W,), jnp.int32)]
            + [pltpu.VMEM((_LANES, DMODEL), jnp.float32)] * 2
            + [pltpu.VMEM((8, 8, _LANES), jnp.float32)] * 2
            + [pltpu.SemaphoreType.DMA] * 4
        ),
        compiler_params=pltpu.CompilerParams(use_tc_tiling_on_sc=False,
                                             needs_layout_passes=False),
    )
    def lookup(idx_hbm, table_hbm, out_hbm,
               idx_v, rows0, rows1, tb0, tb1, g0, g1, s0, s1):
        rows, tbuf = (rows0, rows1), (tb0, tb1)
        gsem, ssem = (g0, g1), (s0, s1)
        wid = lax.axis_index("s") * _NC + lax.axis_index("c")
        jbase = wid * _BPW
        pltpu.sync_copy(idx_hbm.at[pl.ds(jbase * _LANES, _IDX_PER_W)], idx_v)

        iotas = [lax.iota(jnp.int32, 16) + 16 * k for k in range(8)]

        def start_gather(t, b):
            pltpu.async_copy(
                table_hbm.at[idx_v.at[pl.ds(t * _LANES, _LANES)]],
                rows[b], gsem[b])

        def wait_gather(b):
            pltpu.make_async_copy(
                table_hbm.at[idx_v.at[pl.ds(0, _LANES)]],
                rows[b], gsem[b]).wait()

        def start_store(t, b):
            j = jbase + t
            b1 = j // _CTILES
            c = j % _CTILES
            pltpu.async_copy(tbuf[b], out_hbm.at[b1, :, c], ssem[b])

        def wait_store(b):
            pltpu.make_async_copy(tbuf[b], out_hbm.at[0, :, 0],
                                  ssem[b]).wait()

        def transpose_scale(b):
            rb, tb = rows[b], tbuf[b]

            def body_j(j, carry):
                dloc = (iotas[0] + j) & 15
                s_idx = dloc & 7
                rloc = dloc >> 3
                for m in range(DMODEL // 16):
                    d_vec = dloc + 16 * m
                    r_idx = rloc + 2 * m
                    for k in range(8):
                        v = plsc.load_gather(rb, [iotas[k], d_vec])
                        plsc.store_scatter(tb, [r_idx, s_idx, iotas[k]],
                                           v * SCALE)
                return carry

            lax.fori_loop(0, 16, body_j, 0, unroll=False)

        def step(t, b, wait_st, prefetch):
            wait_gather(b)
            if wait_st:
                wait_store(b)
            transpose_scale(b)
            start_store(t, b)
            if prefetch:
                start_gather(t + 2, b)

        start_gather(0, 0)
        start_gather(1, 1)
        step(0, 0, wait_st=False, prefetch=True)
        step(1, 1, wait_st=False, prefetch=True)

        def round_body(i, carry):
            step(2 * i, 0, wait_st=True, prefetch=True)
            step(2 * i + 1, 1, wait_st=True, prefetch=True)
            return carry

        lax.fori_loop(1, _BPW // 2 - 1, round_body, 0, unroll=False)

        step(_BPW - 2, 0, wait_st=True, prefetch=False)
        step(_BPW - 1, 1, wait_st=True, prefetch=False)
        wait_store(0)
        wait_store(1)

    return lookup


def kernel(x, table):
    idx = x.T.reshape(-1).astype(jnp.int32)
    out5 = _make_lookup()(idx, table)
    return out5.transpose(2, 4, 0, 1, 3).reshape(_B0, _B1, DMODEL)
